# baseline (speedup 1.0000x reference)
import functools

import jax
import jax.numpy as jnp
from jax import lax
from jax.experimental import pallas as pl
from jax.experimental.pallas import tpu as pltpu
from jax.experimental.pallas import tpu_sc as plsc

_B = 16384
_GD = 32
_MD = 128
_NC, _NS = 2, 16
_NW = _NC * _NS
_BPW = _B // _NW
_CH = 128
_NCH = _BPW // _CH
_MHALF = _NCH // 2

_MESH = plsc.VectorSubcoreMesh(core_axis_name="c", subcore_axis_name="s")


def _sc_mlp_body(uidx, iidx, umlp, imlp, um_out, im_out,
                 uidx_v, iidx_v, um_v, im_v, sem):
    wid = lax.axis_index("s") * _NC + lax.axis_index("c")
    base = wid * _BPW
    pltpu.sync_copy(uidx.at[pl.ds(base, _BPW)], uidx_v)
    pltpu.sync_copy(iidx.at[pl.ds(base, _BPW)], iidx_v)
    half = _MHALF * _CH
    for h in range(2):
        cps = []
        for k in range(_MHALF):
            off = h * half + k * _CH
            cps.append(pltpu.async_copy(
                umlp.at[uidx_v.at[pl.ds(off, _CH)]],
                um_v.at[pl.ds(k * _CH, _CH)], sem))
            cps.append(pltpu.async_copy(
                imlp.at[iidx_v.at[pl.ds(off, _CH)]],
                im_v.at[pl.ds(k * _CH, _CH)], sem))
        for cp in cps:
            cp.wait()
        pltpu.sync_copy(um_v, um_out.at[pl.ds(base + h * half, half)])
        pltpu.sync_copy(im_v, im_out.at[pl.ds(base + h * half, half)])


_sc_mlp = functools.partial(
    pl.kernel,
    out_type=(
        jax.ShapeDtypeStruct((_B, _MD), jnp.float32),
        jax.ShapeDtypeStruct((_B, _MD), jnp.float32),
    ),
    mesh=_MESH,
    scratch_types=[
        pltpu.VMEM((_BPW,), jnp.int32),
        pltpu.VMEM((_BPW,), jnp.int32),
        pltpu.VMEM((_MHALF * _CH, _MD), jnp.float32),
        pltpu.VMEM((_MHALF * _CH, _MD), jnp.float32),
        pltpu.SemaphoreType.DMA,
    ],
)(_sc_mlp_body)


def _tc_mlp_body(ugT, ig, um, im, w1u, w1i, b1, w2, b2, w3, b3,
                 wfg, wfm, bf, out):
    h = jnp.dot(um[...], w1u[...], preferred_element_type=jnp.float32)
    h = h + jnp.dot(im[...], w1i[...], preferred_element_type=jnp.float32)
    h = jnp.maximum(h + b1[...], 0.0)
    h = jnp.maximum(jnp.dot(h, w2[...], preferred_element_type=jnp.float32) + b2[...], 0.0)
    h = jnp.maximum(jnp.dot(h, w3[...], preferred_element_type=jnp.float32) + b3[...], 0.0)
    g = ugT[...].T * ig[...]
    r = jnp.dot(g, wfg[...], preferred_element_type=jnp.float32)
    r = r + jnp.dot(h, wfm[...], preferred_element_type=jnp.float32)
    out[...] = r[:, 0] + bf[0, 0]


def _tc_mlp(ugT, ig, um, im, w1u, w1i, b1, w2, b2, w3, b3, wfg, wfm, bf):
    blk = 2048
    grid = (_B // blk,)
    fixed = lambda shape: pl.BlockSpec(shape, lambda i: (0,) * len(shape))
    return pl.pallas_call(
        _tc_mlp_body,
        grid=grid,
        in_specs=[
            pl.BlockSpec((_GD, blk), lambda i: (0, i)),
            pl.BlockSpec((blk, _GD), lambda i: (i, 0)),
            pl.BlockSpec((blk, _MD), lambda i: (i, 0)),
            pl.BlockSpec((blk, _MD), lambda i: (i, 0)),
            fixed((_MD, _MD)),
            fixed((_MD, _MD)),
            fixed((1, _MD)),
            fixed((_MD, 64)),
            fixed((1, 64)),
            fixed((64, _GD)),
            fixed((1, _GD)),
            fixed((_GD, 1)),
            fixed((_GD, 1)),
            fixed((1, 1)),
        ],
        out_specs=pl.BlockSpec((blk,), lambda i: (i,)),
        out_shape=jax.ShapeDtypeStruct((_B,), jnp.float32),
    )(ugT, ig, um, im, w1u, w1i, b1, w2, b2, w3, b3, wfg, wfm, bf)


def kernel(user_indices, item_indices, user_gmf_table, item_gmf_table,
           user_mlp_table, item_mlp_table, W1, b1, W2, b2, W3, b3, Wf, bf):
    ugT = jnp.take(user_gmf_table, user_indices, axis=0).T
    ig = jnp.take(item_gmf_table, item_indices, axis=0)
    um, im = _sc_mlp(user_indices, item_indices, user_mlp_table, item_mlp_table)
    w1u = W1[:, :_MD].T
    w1i = W1[:, _MD:].T
    wfg = Wf[:, :_GD].T
    wfm = Wf[:, _GD:].T
    return _tc_mlp(ugT, ig, um, im, w1u, w1i, b1.reshape(1, _MD),
                   W2.T, b2.reshape(1, 64), W3.T, b3.reshape(1, _GD),
                   wfg, wfm, bf.reshape(1, 1))


# speedup vs baseline: 1.0955x; 1.0955x over previous
import functools

import jax
import jax.numpy as jnp
from jax import lax
from jax.experimental import pallas as pl
from jax.experimental.pallas import tpu as pltpu
from jax.experimental.pallas import tpu_sc as plsc

_B = 16384
_GD = 32
_MD = 128
_NC, _NS = 2, 16
_NW = _NC * _NS
_CH = 128
_HALVES = 2
_HB = _B // _HALVES

_MESH = plsc.VectorSubcoreMesh(core_axis_name="c", subcore_axis_name="s")


def _make_sc_mlp(nb):
    bpw = nb // _NW
    nch = bpw // _CH
    mhalf = max(nch // 2, 1)
    nphase = nch // mhalf
    half = mhalf * _CH

    def body(uidx, iidx, umlp, imlp, um_out, im_out,
             uidx_v, iidx_v, um_v, im_v, sem):
        wid = lax.axis_index("s") * _NC + lax.axis_index("c")
        base = wid * bpw
        pltpu.sync_copy(uidx.at[pl.ds(base, bpw)], uidx_v)
        pltpu.sync_copy(iidx.at[pl.ds(base, bpw)], iidx_v)
        for h in range(nphase):
            cps = []
            for k in range(mhalf):
                off = h * half + k * _CH
                cps.append(pltpu.async_copy(
                    umlp.at[uidx_v.at[pl.ds(off, _CH)]],
                    um_v.at[pl.ds(k * _CH, _CH)], sem))
                cps.append(pltpu.async_copy(
                    imlp.at[iidx_v.at[pl.ds(off, _CH)]],
                    im_v.at[pl.ds(k * _CH, _CH)], sem))
            for cp in cps:
                cp.wait()
            pltpu.sync_copy(um_v, um_out.at[pl.ds(base + h * half, half)])
            pltpu.sync_copy(im_v, im_out.at[pl.ds(base + h * half, half)])

    return functools.partial(
        pl.kernel,
        out_type=(
            jax.ShapeDtypeStruct((nb, _MD), jnp.float32),
            jax.ShapeDtypeStruct((nb, _MD), jnp.float32),
        ),
        mesh=_MESH,
        scratch_types=[
            pltpu.VMEM((bpw,), jnp.int32),
            pltpu.VMEM((bpw,), jnp.int32),
            pltpu.VMEM((half, _MD), jnp.float32),
            pltpu.VMEM((half, _MD), jnp.float32),
            pltpu.SemaphoreType.DMA,
        ],
    )(body)


_sc_mlp_half = _make_sc_mlp(_HB)


def _tc_mlp_body(ugT, igT, um, im, w1u, w1i, b1, w2, b2, w3, b3,
                 wfg, wfm, bf, out):
    h = jnp.dot(um[...], w1u[...], preferred_element_type=jnp.float32)
    h = h + jnp.dot(im[...], w1i[...], preferred_element_type=jnp.float32)
    h = jnp.maximum(h + b1[...], 0.0)
    h = jnp.maximum(jnp.dot(h, w2[...], preferred_element_type=jnp.float32) + b2[...], 0.0)
    h = jnp.maximum(jnp.dot(h, w3[...], preferred_element_type=jnp.float32) + b3[...], 0.0)
    gmf = jnp.sum(ugT[...] * igT[...] * wfg[...], axis=0)
    out[...] = gmf + jnp.sum(h * wfm[...], axis=1) + bf[0, 0]


def _tc_mlp(nb, ugT, igT, um, im, w1u, w1i, b1, w2, b2, w3, b3, wfg, wfm, bf):
    blk = 2048
    grid = (nb // blk,)
    fixed = lambda shape: pl.BlockSpec(shape, lambda i: (0,) * len(shape))
    return pl.pallas_call(
        _tc_mlp_body,
        grid=grid,
        in_specs=[
            pl.BlockSpec((_GD, blk), lambda i: (0, i)),
            pl.BlockSpec((_GD, blk), lambda i: (0, i)),
            pl.BlockSpec((blk, _MD), lambda i: (i, 0)),
            pl.BlockSpec((blk, _MD), lambda i: (i, 0)),
            fixed((_MD, _MD)),
            fixed((_MD, _MD)),
            fixed((1, _MD)),
            fixed((_MD, 64)),
            fixed((1, 64)),
            fixed((64, _GD)),
            fixed((1, _GD)),
            fixed((_GD, 1)),
            fixed((1, _GD)),
            fixed((1, 1)),
        ],
        out_specs=pl.BlockSpec((blk,), lambda i: (i,)),
        out_shape=jax.ShapeDtypeStruct((nb,), jnp.float32),
    )(ugT, igT, um, im, w1u, w1i, b1, w2, b2, w3, b3, wfg, wfm, bf)


def kernel(user_indices, item_indices, user_gmf_table, item_gmf_table,
           user_mlp_table, item_mlp_table, W1, b1, W2, b2, W3, b3, Wf, bf):
    w1u = W1[:, :_MD].T
    w1i = W1[:, _MD:].T
    wfg = Wf[:, :_GD].T
    wfm = Wf[:, _GD:]
    b1r = b1.reshape(1, _MD)
    b2r = b2.reshape(1, 64)
    b3r = b3.reshape(1, _GD)
    bfr = bf.reshape(1, 1)
    w2t = W2.T
    w3t = W3.T
    outs = []
    for h in range(_HALVES):
        uidx = lax.dynamic_slice_in_dim(user_indices, h * _HB, _HB)
        iidx = lax.dynamic_slice_in_dim(item_indices, h * _HB, _HB)
        ugT = jnp.take(user_gmf_table, uidx, axis=0).T
        igT = jnp.take(item_gmf_table, iidx, axis=0).T
        um, im = _sc_mlp_half(uidx, iidx, user_mlp_table, item_mlp_table)
        outs.append(_tc_mlp(_HB, ugT, igT, um, im, w1u, w1i, b1r,
                            w2t, b2r, w3t, b3r, wfg, wfm, bfr))
    return jnp.concatenate(outs)


# speedup vs baseline: 1.2342x; 1.1267x over previous
import functools

import jax
import jax.numpy as jnp
from jax import lax
from jax.experimental import pallas as pl
from jax.experimental.pallas import tpu as pltpu
from jax.experimental.pallas import tpu_sc as plsc

_B = 16384
_GD = 32
_MD = 128
_NC, _NS = 2, 16
_NW = _NC * _NS
_BPW = _B // _NW
_CH = 128
_NCH = _BPW // _CH
_MHALF = _NCH // 2

_MESH = plsc.VectorSubcoreMesh(core_axis_name="c", subcore_axis_name="s")


def _sc_mlp_body(uidx, iidx, umlp, imlp, um_out, im_out,
                 uidx_v, iidx_v, um_v, im_v, sem):
    wid = lax.axis_index("s") * _NC + lax.axis_index("c")
    base = wid * _BPW
    pltpu.sync_copy(uidx.at[pl.ds(base, _BPW)], uidx_v)
    pltpu.sync_copy(iidx.at[pl.ds(base, _BPW)], iidx_v)
    half = _MHALF * _CH
    for h in range(2):
        cps = []
        for k in range(_MHALF):
            off = h * half + k * _CH
            cps.append(pltpu.async_copy(
                umlp.at[uidx_v.at[pl.ds(off, _CH)]],
                um_v.at[pl.ds(k * _CH, _CH)], sem))
            cps.append(pltpu.async_copy(
                imlp.at[iidx_v.at[pl.ds(off, _CH)]],
                im_v.at[pl.ds(k * _CH, _CH)], sem))
        for cp in cps:
            cp.wait()
        pltpu.sync_copy(um_v, um_out.at[pl.ds(base + h * half, half)])
        pltpu.sync_copy(im_v, im_out.at[pl.ds(base + h * half, half)])


_sc_mlp = functools.partial(
    pl.kernel,
    out_type=(
        jax.ShapeDtypeStruct((_B, _MD), jnp.float32),
        jax.ShapeDtypeStruct((_B, _MD), jnp.float32),
    ),
    mesh=_MESH,
    scratch_types=[
        pltpu.VMEM((_BPW,), jnp.int32),
        pltpu.VMEM((_BPW,), jnp.int32),
        pltpu.VMEM((_MHALF * _CH, _MD), jnp.float32),
        pltpu.VMEM((_MHALF * _CH, _MD), jnp.float32),
        pltpu.SemaphoreType.DMA,
    ],
)(_sc_mlp_body)


def _tc_mlp_body(ugT, igT, um, im, w1u, w1i, b1, w2, b2, w3, b3,
                 wfg, wfm, bf, out):
    h = jnp.dot(um[...], w1u[...], preferred_element_type=jnp.float32)
    h = h + jnp.dot(im[...], w1i[...], preferred_element_type=jnp.float32)
    h = jnp.maximum(h + b1[...], 0.0)
    h = jnp.maximum(jnp.dot(h, w2[...], preferred_element_type=jnp.float32) + b2[...], 0.0)
    h = jnp.maximum(jnp.dot(h, w3[...], preferred_element_type=jnp.float32) + b3[...], 0.0)
    g = ugT[...] * igT[...]
    gr = jnp.dot(wfg[...], g, preferred_element_type=jnp.float32)
    mr = jnp.dot(h, wfm[...], preferred_element_type=jnp.float32)
    out[...] = gr[0, :] + mr[:, 0] + bf[0, 0]


def _tc_mlp(ugT, igT, um, im, w1u, w1i, b1, w2, b2, w3, b3, wfg, wfm, bf):
    blk = 2048
    grid = (_B // blk,)
    fixed = lambda shape: pl.BlockSpec(shape, lambda i: (0,) * len(shape))
    return pl.pallas_call(
        _tc_mlp_body,
        grid=grid,
        in_specs=[
            pl.BlockSpec((_GD, blk), lambda i: (0, i)),
            pl.BlockSpec((_GD, blk), lambda i: (0, i)),
            pl.BlockSpec((blk, _MD), lambda i: (i, 0)),
            pl.BlockSpec((blk, _MD), lambda i: (i, 0)),
            fixed((_MD, _MD)),
            fixed((_MD, _MD)),
            fixed((1, _MD)),
            fixed((_MD, 64)),
            fixed((1, 64)),
            fixed((64, _GD)),
            fixed((1, _GD)),
            fixed((1, _GD)),
            fixed((_GD, 1)),
            fixed((1, 1)),
        ],
        out_specs=pl.BlockSpec((blk,), lambda i: (i,)),
        out_shape=jax.ShapeDtypeStruct((_B,), jnp.float32),
    )(ugT, igT, um, im, w1u, w1i, b1, w2, b2, w3, b3, wfg, wfm, bf)


def kernel(user_indices, item_indices, user_gmf_table, item_gmf_table,
           user_mlp_table, item_mlp_table, W1, b1, W2, b2, W3, b3, Wf, bf):
    ugT = jnp.take(user_gmf_table, user_indices, axis=0).T
    igT = jnp.take(item_gmf_table.T, item_indices, axis=1)
    um, im = _sc_mlp(user_indices, item_indices, user_mlp_table, item_mlp_table)
    w1u = W1[:, :_MD].T
    w1i = W1[:, _MD:].T
    wfg = Wf[:, :_GD]
    wfm = Wf[:, _GD:].T
    return _tc_mlp(ugT, igT, um, im, w1u, w1i, b1.reshape(1, _MD),
                   W2.T, b2.reshape(1, 64), W3.T, b3.reshape(1, _GD),
                   wfg, wfm, bf.reshape(1, 1))
